# baseline (speedup 1.0000x reference)
import functools

import jax
import jax.numpy as jnp
from jax import lax
from jax.experimental import pallas as pl
from jax.experimental.pallas import tpu as pltpu
from jax.experimental.pallas import tpu_sc as plsc

HIDDEN = 768
LANES = 16
NCH = HIDDEN // LANES
EPS = 1e-5
N_TOKENS = 4 * 2048
NUM_WORKERS = 32
TOK_PER_WORKER = N_TOKENS // NUM_WORKERS
CHUNK = 64
CHUNKS_PER_WORKER = TOK_PER_WORKER // CHUNK


def _body(ids_hbm, pids_hbm, tids_hbm, word_hbm, pos_hbm, tt_hbm,
          scale_hbm, bias_hbm, out_hbm,
          idw_v, idp_v, idt_v, wrows, prows, tt_v, sb_v, sem_w, sem_p):
    wid = lax.axis_index("s") * 2 + lax.axis_index("c")
    base = wid * TOK_PER_WORKER

    pltpu.sync_copy(tt_hbm, tt_v)
    pltpu.sync_copy(scale_hbm, sb_v.at[0])
    pltpu.sync_copy(bias_hbm, sb_v.at[1])
    for j in range(NCH):
        sl = pl.ds(j * LANES, LANES)
        tt_v[1, sl] = tt_v[1, sl] - tt_v[0, sl]

    def chunk_body(c, carry):
        off = base + c * CHUNK
        pltpu.sync_copy(ids_hbm.at[pl.ds(off, CHUNK)], idw_v)
        pltpu.sync_copy(pids_hbm.at[pl.ds(off, CHUNK)], idp_v)
        pltpu.sync_copy(tids_hbm.at[pl.ds(off, CHUNK)], idt_v)
        cw = pltpu.async_copy(word_hbm.at[idw_v], wrows, sem_w)
        cp = pltpu.async_copy(pos_hbm.at[idp_v], prows, sem_p)
        cw.wait()
        cp.wait()

        def token_body(t, tc):
            grp = (t // LANES) * LANES
            lane = t - grp
            tgrp = idt_v[pl.ds(grp, LANES)]
            dnums = lax.GatherDimensionNumbers(
                offset_dims=(), collapsed_slice_dims=(0,), start_index_map=(0,))
            tvec = lax.gather(
                tgrp, jnp.full((LANES, 1), lane, jnp.int32), dnums, (1,),
                mode=lax.GatherScatterMode.PROMISE_IN_BOUNDS)
            s_f = tvec.astype(jnp.float32)

            acc = jnp.zeros((LANES,), jnp.float32)
            accsq = jnp.zeros((LANES,), jnp.float32)
            for j in range(NCH):
                sl = pl.ds(j * LANES, LANES)
                x = wrows[t, sl] + prows[t, sl] + tt_v[0, sl] + s_f * tt_v[1, sl]
                acc = acc + x
                accsq = accsq + x * x
                wrows[t, sl] = x

            tot = jnp.sum(acc)
            totsq = jnp.sum(accsq)
            inv_h = jnp.float32(1.0 / HIDDEN)
            mean_v = jnp.full((LANES,), tot, jnp.float32) * inv_h
            var_v = jnp.full((LANES,), totsq, jnp.float32) * inv_h - mean_v * mean_v
            a = var_v + jnp.float32(EPS)
            seed_i = jnp.int32(0x5F3759DF) - (plsc.bitcast(a, jnp.int32) >> 1)
            y = plsc.bitcast(seed_i, jnp.float32)
            half_a = a * jnp.float32(0.5)
            for _ in range(3):
                y = y * (jnp.float32(1.5) - half_a * y * y)

            for j in range(NCH):
                sl = pl.ds(j * LANES, LANES)
                x = wrows[t, sl]
                wrows[t, sl] = (x - mean_v) * y * sb_v[0, sl] + sb_v[1, sl]
            return tc

        lax.fori_loop(0, CHUNK, token_body, 0)
        pltpu.sync_copy(wrows, out_hbm.at[pl.ds(off, CHUNK)])
        return carry

    lax.fori_loop(0, CHUNKS_PER_WORKER, chunk_body, 0)


@jax.jit
def _run(ids, pids, tids, word_embeddings, position_embeddings,
         token_type_embeddings, ln_scale, ln_bias):
    mesh = plsc.VectorSubcoreMesh(core_axis_name="c", subcore_axis_name="s")
    fn = functools.partial(
        pl.kernel,
        mesh=mesh,
        compiler_params=pltpu.CompilerParams(needs_layout_passes=False),
        out_type=jax.ShapeDtypeStruct((N_TOKENS, HIDDEN), jnp.float32),
        scratch_types=[
            pltpu.VMEM((CHUNK,), jnp.int32),
            pltpu.VMEM((CHUNK,), jnp.int32),
            pltpu.VMEM((CHUNK,), jnp.int32),
            pltpu.VMEM((CHUNK, HIDDEN), jnp.float32),
            pltpu.VMEM((CHUNK, HIDDEN), jnp.float32),
            pltpu.VMEM((2, HIDDEN), jnp.float32),
            pltpu.VMEM((2, HIDDEN), jnp.float32),
            pltpu.SemaphoreType.DMA,
            pltpu.SemaphoreType.DMA,
        ],
    )(_body)
    return fn(ids, pids, tids, word_embeddings, position_embeddings,
              token_type_embeddings, ln_scale, ln_bias)


def kernel(input_ids, token_type_ids, position_ids, attention_mask,
           word_embeddings, position_embeddings, token_type_embeddings,
           ln_scale, ln_bias):
    del attention_mask
    ids = input_ids.reshape(-1).astype(jnp.int32)
    pids = position_ids.reshape(-1).astype(jnp.int32)
    tids = token_type_ids.reshape(-1).astype(jnp.int32)
    out = _run(ids, pids, tids, word_embeddings, position_embeddings,
               token_type_embeddings, ln_scale, ln_bias)
    return out.reshape(input_ids.shape + (HIDDEN,))


# speedup vs baseline: 2.5186x; 2.5186x over previous
import functools

import jax
import jax.numpy as jnp
from jax import lax
from jax.experimental import pallas as pl
from jax.experimental.pallas import tpu as pltpu
from jax.experimental.pallas import tpu_sc as plsc

HIDDEN = 768
LANES = 16
NCH = HIDDEN // LANES
EPS = 1e-5
N_TOKENS = 4 * 2048
NUM_WORKERS = 32
TOK_PER_WORKER = N_TOKENS // NUM_WORKERS
CHUNK = 32
NCHUNKS = TOK_PER_WORKER // CHUNK
ROW_BLOCK = 512


def _gather_sum_body(ids_hbm, pids_hbm, word_hbm, pos_hbm, x_hbm,
                     idw0, idw1, idp0, idp1, w0, w1, p0, p1,
                     sw0, sw1, sp0, sp1, so0, so1):
    wid = lax.axis_index("s") * 2 + lax.axis_index("c")
    base = wid * TOK_PER_WORKER

    idw = (idw0, idw1)
    idp = (idp0, idp1)
    wr = (w0, w1)
    pr = (p0, p1)
    sw = (sw0, sw1)
    sp = (sp0, sp1)
    so = (so0, so1)

    gather_h = [None, None]
    out_h = [None, None]

    def start_gather(c):
        b = c % 2
        off = base + c * CHUNK
        pltpu.sync_copy(ids_hbm.at[pl.ds(off, CHUNK)], idw[b])
        pltpu.sync_copy(pids_hbm.at[pl.ds(off, CHUNK)], idp[b])
        hw = pltpu.async_copy(word_hbm.at[idw[b]], wr[b], sw[b])
        hp = pltpu.async_copy(pos_hbm.at[idp[b]], pr[b], sp[b])
        gather_h[b] = (hw, hp)

    start_gather(0)
    for c in range(NCHUNKS):
        b = c % 2
        hw, hp = gather_h[b]
        hw.wait()
        hp.wait()
        if c + 1 < NCHUNKS:
            if out_h[1 - b] is not None:
                out_h[1 - b].wait()
            start_gather(c + 1)

        wb, pb = wr[b], pr[b]

        def sum_body(t, carry, wb=wb, pb=pb):
            for j in range(NCH):
                sl = pl.ds(j * LANES, LANES)
                wb[t, sl] = wb[t, sl] + pb[t, sl]
            return carry

        lax.fori_loop(0, CHUNK, sum_body, 0)
        off = base + c * CHUNK
        out_h[b] = pltpu.async_copy(wb, x_hbm.at[pl.ds(off, CHUNK)], so[b])
    for b in (0, 1):
        if out_h[b] is not None:
            out_h[b].wait()


@jax.jit
def _gather_sum(ids, pids, word_embeddings, position_embeddings):
    mesh = plsc.VectorSubcoreMesh(core_axis_name="c", subcore_axis_name="s")
    fn = functools.partial(
        pl.kernel,
        mesh=mesh,
        compiler_params=pltpu.CompilerParams(needs_layout_passes=False),
        out_type=jax.ShapeDtypeStruct((N_TOKENS, HIDDEN), jnp.float32),
        scratch_types=[
            pltpu.VMEM((CHUNK,), jnp.int32),
            pltpu.VMEM((CHUNK,), jnp.int32),
            pltpu.VMEM((CHUNK,), jnp.int32),
            pltpu.VMEM((CHUNK,), jnp.int32),
            pltpu.VMEM((CHUNK, HIDDEN), jnp.float32),
            pltpu.VMEM((CHUNK, HIDDEN), jnp.float32),
            pltpu.VMEM((CHUNK, HIDDEN), jnp.float32),
            pltpu.VMEM((CHUNK, HIDDEN), jnp.float32),
            pltpu.SemaphoreType.DMA,
            pltpu.SemaphoreType.DMA,
            pltpu.SemaphoreType.DMA,
            pltpu.SemaphoreType.DMA,
            pltpu.SemaphoreType.DMA,
            pltpu.SemaphoreType.DMA,
        ],
    )(_gather_sum_body)
    return fn(ids, pids, word_embeddings, position_embeddings)


def _ln_body(tidf_ref, tt_ref, scale_ref, bias_ref, x_ref, o_ref):
    x = x_ref[...]
    t0 = tt_ref[0:1, :]
    d = tt_ref[1:2, :] - t0
    x = x + t0 + tidf_ref[...] * d
    mean = jnp.mean(x, axis=1, keepdims=True)
    xc = x - mean
    var = jnp.mean(xc * xc, axis=1, keepdims=True)
    y = xc * lax.rsqrt(var + EPS)
    o_ref[...] = y * scale_ref[...] + bias_ref[...]


@jax.jit
def _type_ln(x, tidf, token_type_embeddings, scale2d, bias2d):
    grid = (N_TOKENS // ROW_BLOCK,)
    return pl.pallas_call(
        _ln_body,
        grid=grid,
        in_specs=[
            pl.BlockSpec((ROW_BLOCK, 1), lambda i: (i, 0)),
            pl.BlockSpec((2, HIDDEN), lambda i: (0, 0)),
            pl.BlockSpec((1, HIDDEN), lambda i: (0, 0)),
            pl.BlockSpec((1, HIDDEN), lambda i: (0, 0)),
            pl.BlockSpec((ROW_BLOCK, HIDDEN), lambda i: (i, 0)),
        ],
        out_specs=pl.BlockSpec((ROW_BLOCK, HIDDEN), lambda i: (i, 0)),
        out_shape=jax.ShapeDtypeStruct((N_TOKENS, HIDDEN), jnp.float32),
    )(tidf, token_type_embeddings, scale2d, bias2d, x)


def kernel(input_ids, token_type_ids, position_ids, attention_mask,
           word_embeddings, position_embeddings, token_type_embeddings,
           ln_scale, ln_bias):
    del attention_mask
    ids = input_ids.reshape(-1).astype(jnp.int32)
    pids = position_ids.reshape(-1).astype(jnp.int32)
    tidf = token_type_ids.reshape(-1, 1).astype(jnp.float32)
    x = _gather_sum(ids, pids, word_embeddings, position_embeddings)
    out = _type_ln(x, tidf, token_type_embeddings,
                   ln_scale.reshape(1, HIDDEN), ln_bias.reshape(1, HIDDEN))
    return out.reshape(input_ids.shape + (HIDDEN,))


# speedup vs baseline: 2.6330x; 1.0454x over previous
import functools

import jax
import jax.numpy as jnp
from jax import lax
from jax.experimental import pallas as pl
from jax.experimental.pallas import tpu as pltpu
from jax.experimental.pallas import tpu_sc as plsc

HIDDEN = 768
LANES = 16
NCH = HIDDEN // LANES
EPS = 1e-5
N_TOKENS = 4 * 2048
NUM_WORKERS = 32
TOK_PER_WORKER = N_TOKENS // NUM_WORKERS
CHUNK = 32
NCHUNKS = TOK_PER_WORKER // CHUNK
ROW_BLOCK = 1024


def _gather_sum_body(ids_hbm, pids_hbm, word_hbm, pos_hbm, x_hbm,
                     idw0, idw1, idp0, idp1, w0, w1, p0, p1,
                     sw0, sw1, sp0, sp1, so0, so1):
    wid = lax.axis_index("s") * 2 + lax.axis_index("c")
    base = wid * TOK_PER_WORKER

    idw = (idw0, idw1)
    idp = (idp0, idp1)
    wr = (w0, w1)
    pr = (p0, p1)
    sw = (sw0, sw1)
    sp = (sp0, sp1)
    so = (so0, so1)

    gather_h = [None, None]
    out_h = [None, None]

    def start_gather(c):
        b = c % 2
        off = base + c * CHUNK
        pltpu.sync_copy(ids_hbm.at[pl.ds(off, CHUNK)], idw[b])
        pltpu.sync_copy(pids_hbm.at[pl.ds(off, CHUNK)], idp[b])
        hw = pltpu.async_copy(word_hbm.at[idw[b]], wr[b], sw[b])
        hp = pltpu.async_copy(pos_hbm.at[idp[b]], pr[b], sp[b])
        gather_h[b] = (hw, hp)

    start_gather(0)
    for c in range(NCHUNKS):
        b = c % 2
        hw, hp = gather_h[b]
        hw.wait()
        hp.wait()
        if c + 1 < NCHUNKS:
            if out_h[1 - b] is not None:
                out_h[1 - b].wait()
            start_gather(c + 1)

        wb, pb = wr[b], pr[b]

        def sum_body(t, carry, wb=wb, pb=pb):
            for j in range(NCH):
                sl = pl.ds(j * LANES, LANES)
                wb[t, sl] = wb[t, sl] + pb[t, sl]
            return carry

        lax.fori_loop(0, CHUNK, sum_body, 0)
        off = base + c * CHUNK
        out_h[b] = pltpu.async_copy(wb, x_hbm.at[pl.ds(off, CHUNK)], so[b])
    for b in (0, 1):
        if out_h[b] is not None:
            out_h[b].wait()


@jax.jit
def _gather_sum(ids, pids, word_embeddings, position_embeddings):
    mesh = plsc.VectorSubcoreMesh(core_axis_name="c", subcore_axis_name="s")
    fn = functools.partial(
        pl.kernel,
        mesh=mesh,
        compiler_params=pltpu.CompilerParams(needs_layout_passes=False),
        out_type=jax.ShapeDtypeStruct((N_TOKENS, HIDDEN), jnp.float32),
        scratch_types=[
            pltpu.VMEM((CHUNK,), jnp.int32),
            pltpu.VMEM((CHUNK,), jnp.int32),
            pltpu.VMEM((CHUNK,), jnp.int32),
            pltpu.VMEM((CHUNK,), jnp.int32),
            pltpu.VMEM((CHUNK, HIDDEN), jnp.float32),
            pltpu.VMEM((CHUNK, HIDDEN), jnp.float32),
            pltpu.VMEM((CHUNK, HIDDEN), jnp.float32),
            pltpu.VMEM((CHUNK, HIDDEN), jnp.float32),
            pltpu.SemaphoreType.DMA,
            pltpu.SemaphoreType.DMA,
            pltpu.SemaphoreType.DMA,
            pltpu.SemaphoreType.DMA,
            pltpu.SemaphoreType.DMA,
            pltpu.SemaphoreType.DMA,
        ],
    )(_gather_sum_body)
    return fn(ids, pids, word_embeddings, position_embeddings)


def _ln_body(tidf_ref, tt_ref, scale_ref, bias_ref, x_ref, o_ref):
    x = x_ref[...]
    t0 = tt_ref[0:1, :]
    d = tt_ref[1:2, :] - t0
    x = x + t0 + tidf_ref[...] * d
    mean = jnp.mean(x, axis=1, keepdims=True)
    xc = x - mean
    var = jnp.mean(xc * xc, axis=1, keepdims=True)
    y = xc * lax.rsqrt(var + EPS)
    o_ref[...] = y * scale_ref[...] + bias_ref[...]


@jax.jit
def _type_ln(x, tidf, token_type_embeddings, scale2d, bias2d):
    grid = (N_TOKENS // ROW_BLOCK,)
    return pl.pallas_call(
        _ln_body,
        grid=grid,
        in_specs=[
            pl.BlockSpec((ROW_BLOCK, 1), lambda i: (i, 0)),
            pl.BlockSpec((2, HIDDEN), lambda i: (0, 0)),
            pl.BlockSpec((1, HIDDEN), lambda i: (0, 0)),
            pl.BlockSpec((1, HIDDEN), lambda i: (0, 0)),
            pl.BlockSpec((ROW_BLOCK, HIDDEN), lambda i: (i, 0)),
        ],
        out_specs=pl.BlockSpec((ROW_BLOCK, HIDDEN), lambda i: (i, 0)),
        out_shape=jax.ShapeDtypeStruct((N_TOKENS, HIDDEN), jnp.float32),
    )(tidf, token_type_embeddings, scale2d, bias2d, x)


def kernel(input_ids, token_type_ids, position_ids, attention_mask,
           word_embeddings, position_embeddings, token_type_embeddings,
           ln_scale, ln_bias):
    del attention_mask
    ids = input_ids.reshape(-1).astype(jnp.int32)
    pids = position_ids.reshape(-1).astype(jnp.int32)
    tidf = token_type_ids.reshape(-1, 1).astype(jnp.float32)
    x = _gather_sum(ids, pids, word_embeddings, position_embeddings)
    out = _type_ln(x, tidf, token_type_embeddings,
                   ln_scale.reshape(1, HIDDEN), ln_bias.reshape(1, HIDDEN))
    return out.reshape(input_ids.shape + (HIDDEN,))
